# baseline (speedup 1.0000x reference)
import functools

import jax
import jax.numpy as jnp
from jax import lax
from jax.experimental import pallas as pl
from jax.experimental.pallas import tpu as pltpu
from jax.experimental.pallas import tpu_sc as plsc

N = 10000
R = 24
D = 128
E = 320000

NC = 2
NS = 16
LANE = 16

EPT = E // (NC * NS)
CHUNK = 128
NCH = (EPT + CHUNK - 1) // CHUNK
SLOTS = NCH * CHUNK
E_PAD = 320256

SEG = R * N
CNT_SIZE = SEG + 128
CNT_PER_TILE = CNT_SIZE // NS

OUT_ROWS = 10112
ROWS_PER_TILE = OUT_ROWS // NS

BLK = 2000
NB = N // BLK


def _ymm_body(x_ref, w_ref, y_ref):
    y_ref[0] = jnp.dot(x_ref[...], w_ref[0], preferred_element_type=jnp.float32)


def _relation_transform(x, w):
    return pl.pallas_call(
        _ymm_body,
        grid=(R, NB),
        in_specs=[
            pl.BlockSpec((BLK, D), lambda r, b: (b, 0)),
            pl.BlockSpec((1, D, D), lambda r, b: (r, 0, 0)),
        ],
        out_specs=pl.BlockSpec((1, BLK, D), lambda r, b: (r, b, 0)),
        out_shape=jax.ShapeDtypeStruct((R, N, D), jnp.float32),
    )(x, w)


def _fin_body(x_ref, root_ref, bias_ref, p0_ref, p1_ref, o_ref):
    x = x_ref[...]
    o_ref[...] = (
        x
        + jnp.dot(x, root_ref[...], preferred_element_type=jnp.float32)
        + bias_ref[...]
        + p0_ref[0]
        + p1_ref[0]
    )


def _combine(x, root, bias2d, partials):
    return pl.pallas_call(
        _fin_body,
        grid=(NB,),
        in_specs=[
            pl.BlockSpec((BLK, D), lambda b: (b, 0)),
            pl.BlockSpec((D, D), lambda b: (0, 0)),
            pl.BlockSpec((1, D), lambda b: (0, 0)),
            pl.BlockSpec((1, BLK, D), lambda b: (0, b, 0)),
            pl.BlockSpec((1, BLK, D), lambda b: (1, b, 0)),
        ],
        out_specs=pl.BlockSpec((BLK, D), lambda b: (b, 0)),
        out_shape=jax.ShapeDtypeStruct((N, D), jnp.float32),
    )(x, root, bias2d, partials, partials)


ZCHUNK = 1072
BATCH = 1024
NBAT = 10
BCH = BATCH // CHUNK
SUB = 64
NSUB = BATCH // SUB


def _sc_body(src_r, dst_r, typ_r, y_r, part_r,
             cnt_sp, acc_sp,
             sstage, dstage, tstage, combA, dstidx, yidxst, normst, cvbuf,
             rb0, rb1, rb2, zlin, onesb, sem, semg, semsc):
    c = lax.axis_index("c")
    s = lax.axis_index("s")
    zero16 = jnp.zeros((LANE,), jnp.float32)
    one16 = jnp.ones((LANE,), jnp.float32)
    lane = lax.broadcasted_iota(jnp.int32, (LANE,), 0)

    def _zrow(i, _):
        for j in range(D // LANE):
            rb0[i, pl.ds(j * LANE, LANE)] = zero16
        return 0
    lax.fori_loop(0, SUB, _zrow, 0)

    def _zlinf(i, _):
        zlin[pl.ds(i * LANE, LANE)] = zero16
        return 0
    lax.fori_loop(0, ZCHUNK // LANE, _zlinf, 0)

    for g in range(CHUNK // LANE):
        onesb[pl.ds(g * LANE, LANE)] = one16

    for k in range(CNT_PER_TILE // ZCHUNK):
        pltpu.sync_copy(
            zlin, cnt_sp.at[pl.ds(s * CNT_PER_TILE + k * ZCHUNK, ZCHUNK)])
    for k in range(ROWS_PER_TILE // SUB):
        pltpu.sync_copy(
            rb0, acc_sp.at[pl.ds(s * ROWS_PER_TILE + k * SUB, SUB), :])
    _rem = ROWS_PER_TILE % SUB
    if _rem:
        pltpu.sync_copy(
            rb0.at[pl.ds(0, _rem), :],
            acc_sp.at[pl.ds(s * ROWS_PER_TILE + ROWS_PER_TILE - _rem, _rem), :])
    plsc.subcore_barrier()

    for p in range(NC):
        base_a = p * (E // NC) + s * EPT

        def _cnt_batch(b, _):
            boff = b * BATCH
            da = pltpu.async_copy(
                typ_r.at[pl.ds(base_a + boff, BATCH)], tstage, sem)
            db = pltpu.async_copy(
                dst_r.at[pl.ds(base_a + boff, BATCH)], dstage, sem)
            da.wait()
            db.wait()

            def _cchunk(ci, _):
                for g in range(CHUNK // LANE):
                    off_l = ci * CHUNK + g * LANE
                    t = tstage[pl.ds(off_l, LANE)]
                    d = dstage[pl.ds(off_l, LANE)]
                    valid = (boff + off_l + lane) < EPT
                    combA[ci, pl.ds(g * LANE, LANE)] = jnp.where(
                        valid, t * N + d, SEG)
                return 0
            lax.fori_loop(0, BCH, _cchunk, 0)
            descs = [
                pltpu.async_copy(
                    onesb, cnt_sp.at[combA.at[ci]], semsc, add=True)
                for ci in range(BCH)
            ]
            for dsc in descs:
                dsc.wait()
            return 0
        with jax.named_scope("phA"):
            lax.fori_loop(0, NBAT, _cnt_batch, 0)
    plsc.subcore_barrier()

    base_c = (s * NC + c) * EPT

    def _edge_batch(b, _):
        boff = b * BATCH
        d1 = pltpu.async_copy(
            src_r.at[pl.ds(base_c + boff, BATCH)], sstage, sem)
        d2 = pltpu.async_copy(
            dst_r.at[pl.ds(base_c + boff, BATCH)], dstage, sem)
        d3 = pltpu.async_copy(
            typ_r.at[pl.ds(base_c + boff, BATCH)], tstage, sem)
        d1.wait()
        d2.wait()
        d3.wait()

        def _cchunk(ci, _):
            for g in range(CHUNK // LANE):
                off_l = ci * CHUNK + g * LANE
                t = tstage[pl.ds(off_l, LANE)]
                d = dstage[pl.ds(off_l, LANE)]
                valid = (boff + off_l + lane) < EPT
                combA[ci, pl.ds(g * LANE, LANE)] = jnp.where(
                    valid, t * N + d, SEG)
            return 0
        lax.fori_loop(0, BCH, _cchunk, 0)
        gds = [
            pltpu.async_copy(cnt_sp.at[combA.at[ci]], cvbuf.at[ci], sem)
            for ci in range(BCH)
        ]
        for gd in gds:
            gd.wait()

        def _nchunk(ci, _):
            for g in range(CHUNK // LANE):
                off_l = ci * CHUNK + g * LANE
                cv = cvbuf[ci, pl.ds(g * LANE, LANE)]
                t = tstage[pl.ds(off_l, LANE)]
                sv = sstage[pl.ds(off_l, LANE)]
                d = dstage[pl.ds(off_l, LANE)]
                valid = (boff + off_l + lane) < EPT
                normst[pl.ds(off_l, LANE)] = jnp.where(
                    valid, 1.0 / jnp.maximum(cv, 1.0), 0.0)
                yidxst[pl.ds(off_l, LANE)] = jnp.where(valid, t * N + sv, 0)
                k2 = off_l // SUB
                dstidx[k2, pl.ds((off_l % SUB) + 0, LANE)] = jnp.where(
                    valid, d, N)
            return 0
        lax.fori_loop(0, BCH, _nchunk, 0)

        rbs = (rb0, rb1, rb2)
        gd = {}
        sd = {}
        for k in range(min(3, NSUB)):
            gd[k] = pltpu.async_copy(
                y_r.at[yidxst.at[pl.ds(k * SUB, SUB)]], rbs[k % 3], semg)
        for k in range(NSUB):
            cur = rbs[k % 3]
            if k >= 1 and (k + 2) < NSUB:
                sd[k - 1].wait()
                gd[k + 2] = pltpu.async_copy(
                    y_r.at[yidxst.at[pl.ds((k + 2) * SUB, SUB)]],
                    rbs[(k + 2) % 3], semg)
            gd[k].wait()

            def _scale(e, _, _cur=cur, _k=k):
                nb = plsc.load_gather(
                    normst, [jnp.full((LANE,), _k * SUB + e, jnp.int32)])
                for j in range(D // LANE):
                    _cur[e, pl.ds(j * LANE, LANE)] = (
                        _cur[e, pl.ds(j * LANE, LANE)] * nb)
                return 0
            lax.fori_loop(0, SUB, _scale, 0)

            sd[k] = pltpu.async_copy(
                cur, acc_sp.at[dstidx.at[k]], semsc, add=True)
        for k in range(max(0, NSUB - 3), NSUB):
            sd[k].wait()
        return 0
    with jax.named_scope("phBC"):
        lax.fori_loop(0, NBAT, _edge_batch, 0)
    plsc.subcore_barrier()

    for k in range(ROWS_PER_TILE // SUB):
        row0 = s * ROWS_PER_TILE + k * SUB
        pltpu.sync_copy(acc_sp.at[pl.ds(row0, SUB), :], rb0)
        pltpu.sync_copy(rb0, part_r.at[c, pl.ds(row0, SUB), :])
    if ROWS_PER_TILE % SUB:
        _rem2 = ROWS_PER_TILE % SUB
        row0 = s * ROWS_PER_TILE + ROWS_PER_TILE - _rem2
        pltpu.sync_copy(acc_sp.at[pl.ds(row0, _rem2), :], rb0.at[pl.ds(0, _rem2), :])
        pltpu.sync_copy(rb0.at[pl.ds(0, _rem2), :], part_r.at[c, pl.ds(row0, _rem2), :])


_sc_scatter = pl.kernel(
    _sc_body,
    out_type=jax.ShapeDtypeStruct((NC, OUT_ROWS, D), jnp.float32),
    mesh=plsc.VectorSubcoreMesh(
        core_axis_name="c", subcore_axis_name="s",
        num_cores=NC, num_subcores=NS),
    compiler_params=pltpu.CompilerParams(needs_layout_passes=False),
    scratch_types=[
        pltpu.VMEM_SHARED((CNT_SIZE,), jnp.float32),
        pltpu.VMEM_SHARED((OUT_ROWS, D), jnp.float32),
        pltpu.VMEM((BATCH,), jnp.int32),
        pltpu.VMEM((BATCH,), jnp.int32),
        pltpu.VMEM((BATCH,), jnp.int32),
        pltpu.VMEM((BCH, CHUNK), jnp.int32),
        pltpu.VMEM((NSUB, SUB), jnp.int32),
        pltpu.VMEM((BATCH,), jnp.int32),
        pltpu.VMEM((BATCH,), jnp.float32),
        pltpu.VMEM((BCH, CHUNK), jnp.float32),
        pltpu.VMEM((SUB, D), jnp.float32),
        pltpu.VMEM((SUB, D), jnp.float32),
        pltpu.VMEM((SUB, D), jnp.float32),
        pltpu.VMEM((ZCHUNK,), jnp.float32),
        pltpu.VMEM((CHUNK,), jnp.float32),
        pltpu.SemaphoreType.DMA,
        pltpu.SemaphoreType.DMA,
        pltpu.SemaphoreType.DMA,
    ],
)


def kernel(node_embeds, weight, root, bias, edge_index, edge_type):
    x = node_embeds.astype(jnp.float32)
    src = edge_index[0].astype(jnp.int32)
    dst = edge_index[1].astype(jnp.int32)
    typ = edge_type.astype(jnp.int32)
    pad = E_PAD - E
    src_p = jnp.concatenate([src, jnp.zeros((pad,), jnp.int32)])
    dst_p = jnp.concatenate([dst, jnp.zeros((pad,), jnp.int32)])
    typ_p = jnp.concatenate([typ, jnp.full((pad,), R, jnp.int32)])

    y = _relation_transform(x, weight).reshape(R * N, D)
    partials = _sc_scatter(src_p, dst_p, typ_p, y)
    bias2d = bias.reshape(1, D)
    return _combine(x, root, bias2d, partials)


# speedup vs baseline: 1.0004x; 1.0004x over previous
import functools

import jax
import jax.numpy as jnp
from jax import lax
from jax.experimental import pallas as pl
from jax.experimental.pallas import tpu as pltpu
from jax.experimental.pallas import tpu_sc as plsc

N = 10000
R = 24
D = 128
E = 320000

NC = 2
NS = 16
LANE = 16

EPT = E // (NC * NS)
CHUNK = 128
NCH = (EPT + CHUNK - 1) // CHUNK
SLOTS = NCH * CHUNK
E_PAD = 320256

SEG = R * N
CNT_SIZE = SEG + 128
CNT_PER_TILE = CNT_SIZE // NS

OUT_ROWS = 10112
ROWS_PER_TILE = OUT_ROWS // NS

BLK = 2000
NB = N // BLK


def _ymm_body(x_ref, w_ref, y_ref):
    y_ref[0] = jnp.dot(x_ref[...], w_ref[0], preferred_element_type=jnp.float32)


def _relation_transform(x, w):
    return pl.pallas_call(
        _ymm_body,
        grid=(R, NB),
        in_specs=[
            pl.BlockSpec((BLK, D), lambda r, b: (b, 0)),
            pl.BlockSpec((1, D, D), lambda r, b: (r, 0, 0)),
        ],
        out_specs=pl.BlockSpec((1, BLK, D), lambda r, b: (r, b, 0)),
        out_shape=jax.ShapeDtypeStruct((R, N, D), jnp.float32),
    )(x, w)


def _fin_body(x_ref, root_ref, bias_ref, p0_ref, p1_ref, o_ref):
    x = x_ref[...]
    o_ref[...] = (
        x
        + jnp.dot(x, root_ref[...], preferred_element_type=jnp.float32)
        + bias_ref[...]
        + p0_ref[0]
        + p1_ref[0]
    )


def _combine(x, root, bias2d, partials):
    return pl.pallas_call(
        _fin_body,
        grid=(NB,),
        in_specs=[
            pl.BlockSpec((BLK, D), lambda b: (b, 0)),
            pl.BlockSpec((D, D), lambda b: (0, 0)),
            pl.BlockSpec((1, D), lambda b: (0, 0)),
            pl.BlockSpec((1, BLK, D), lambda b: (0, b, 0)),
            pl.BlockSpec((1, BLK, D), lambda b: (1, b, 0)),
        ],
        out_specs=pl.BlockSpec((BLK, D), lambda b: (b, 0)),
        out_shape=jax.ShapeDtypeStruct((N, D), jnp.float32),
    )(x, root, bias2d, partials, partials)


ZCHUNK = 1072
BATCH = 1024
NBAT = 10
BCH = BATCH // CHUNK
SUB = 64
NSUB = BATCH // SUB


def _sc_body(src_r, dst_r, typ_r, y_r, part_r,
             cnt_sp, acc_sp,
             sstage, dstage, tstage, combA, dstidx, yidxst, normst, cvbuf,
             rb0, rb1, rb2, zlin, onesb, sem, semg, semsc):
    c = lax.axis_index("c")
    s = lax.axis_index("s")
    zero16 = jnp.zeros((LANE,), jnp.float32)
    one16 = jnp.ones((LANE,), jnp.float32)
    lane = lax.broadcasted_iota(jnp.int32, (LANE,), 0)

    def _zrow(i, _):
        for j in range(D // LANE):
            rb0[i, pl.ds(j * LANE, LANE)] = zero16
        return 0
    lax.fori_loop(0, SUB, _zrow, 0)

    def _zlinf(i, _):
        zlin[pl.ds(i * LANE, LANE)] = zero16
        return 0
    lax.fori_loop(0, ZCHUNK // LANE, _zlinf, 0)

    for g in range(CHUNK // LANE):
        onesb[pl.ds(g * LANE, LANE)] = one16

    for k in range(CNT_PER_TILE // ZCHUNK):
        pltpu.sync_copy(
            zlin, cnt_sp.at[pl.ds(s * CNT_PER_TILE + k * ZCHUNK, ZCHUNK)])
    for k in range(ROWS_PER_TILE // SUB):
        pltpu.sync_copy(
            rb0, acc_sp.at[pl.ds(s * ROWS_PER_TILE + k * SUB, SUB), :])
    _rem = ROWS_PER_TILE % SUB
    if _rem:
        pltpu.sync_copy(
            rb0.at[pl.ds(0, _rem), :],
            acc_sp.at[pl.ds(s * ROWS_PER_TILE + ROWS_PER_TILE - _rem, _rem), :])
    plsc.subcore_barrier()

    for p in range(NC):
        base_a = p * (E // NC) + s * EPT

        def _cnt_batch(b, _):
            boff = b * BATCH
            da = pltpu.async_copy(
                typ_r.at[pl.ds(base_a + boff, BATCH)], tstage, sem)
            db = pltpu.async_copy(
                dst_r.at[pl.ds(base_a + boff, BATCH)], dstage, sem)
            da.wait()
            db.wait()

            def _cchunk(ci, _):
                for g in range(CHUNK // LANE):
                    off_l = ci * CHUNK + g * LANE
                    t = tstage[pl.ds(off_l, LANE)]
                    d = dstage[pl.ds(off_l, LANE)]
                    valid = (boff + off_l + lane) < EPT
                    combA[ci, pl.ds(g * LANE, LANE)] = jnp.where(
                        valid, t * N + d, SEG)
                return 0
            lax.fori_loop(0, BCH, _cchunk, 0)
            descs = [
                pltpu.async_copy(
                    onesb, cnt_sp.at[combA.at[ci]], semsc, add=True)
                for ci in range(BCH)
            ]
            for dsc in descs:
                dsc.wait()
            return 0
        with jax.named_scope("phA"):
            lax.fori_loop(0, NBAT, _cnt_batch, 0)
    plsc.subcore_barrier()

    base_c = (s * NC + c) * EPT

    def _edge_batch(b, _):
      boff = b * BATCH
      with jax.named_scope("bidx"):
        d1 = pltpu.async_copy(
            src_r.at[pl.ds(base_c + boff, BATCH)], sstage, sem)
        d2 = pltpu.async_copy(
            dst_r.at[pl.ds(base_c + boff, BATCH)], dstage, sem)
        d3 = pltpu.async_copy(
            typ_r.at[pl.ds(base_c + boff, BATCH)], tstage, sem)
        d1.wait()
        d2.wait()
        d3.wait()

        def _cchunk(ci, _):
            for g in range(CHUNK // LANE):
                off_l = ci * CHUNK + g * LANE
                t = tstage[pl.ds(off_l, LANE)]
                d = dstage[pl.ds(off_l, LANE)]
                valid = (boff + off_l + lane) < EPT
                combA[ci, pl.ds(g * LANE, LANE)] = jnp.where(
                    valid, t * N + d, SEG)
            return 0
        lax.fori_loop(0, BCH, _cchunk, 0)
        gds = [
            pltpu.async_copy(cnt_sp.at[combA.at[ci]], cvbuf.at[ci], sem)
            for ci in range(BCH)
        ]
        for gd in gds:
            gd.wait()

        def _nchunk(ci, _):
            for g in range(CHUNK // LANE):
                off_l = ci * CHUNK + g * LANE
                cv = cvbuf[ci, pl.ds(g * LANE, LANE)]
                t = tstage[pl.ds(off_l, LANE)]
                sv = sstage[pl.ds(off_l, LANE)]
                d = dstage[pl.ds(off_l, LANE)]
                valid = (boff + off_l + lane) < EPT
                normst[pl.ds(off_l, LANE)] = jnp.where(
                    valid, 1.0 / jnp.maximum(cv, 1.0), 0.0)
                yidxst[pl.ds(off_l, LANE)] = jnp.where(valid, t * N + sv, 0)
                k2 = off_l // SUB
                dstidx[k2, pl.ds((off_l % SUB) + 0, LANE)] = jnp.where(
                    valid, d, N)
            return 0
        lax.fori_loop(0, BCH, _nchunk, 0)

      with jax.named_scope("rows"):
        rbs = (rb0, rb1, rb2)
        gd = {}
        sd = {}
        for k in range(min(3, NSUB)):
            gd[k] = pltpu.async_copy(
                y_r.at[yidxst.at[pl.ds(k * SUB, SUB)]], rbs[k % 3], semg)
        for k in range(NSUB):
            cur = rbs[k % 3]
            if k >= 1 and (k + 2) < NSUB:
                sd[k - 1].wait()
                gd[k + 2] = pltpu.async_copy(
                    y_r.at[yidxst.at[pl.ds((k + 2) * SUB, SUB)]],
                    rbs[(k + 2) % 3], semg)
            gd[k].wait()

            def _scale(e, _, _cur=cur, _k=k):
                nb = plsc.load_gather(
                    normst, [jnp.full((LANE,), _k * SUB + e, jnp.int32)])
                for j in range(D // LANE):
                    _cur[e, pl.ds(j * LANE, LANE)] = (
                        _cur[e, pl.ds(j * LANE, LANE)] * nb)
                return 0
            lax.fori_loop(0, SUB, _scale, 0)

            sd[k] = pltpu.async_copy(
                cur, acc_sp.at[dstidx.at[k]], semsc, add=True)
        for k in range(max(0, NSUB - 3), NSUB):
            sd[k].wait()
      return 0
    with jax.named_scope("phBC"):
        lax.fori_loop(0, NBAT, _edge_batch, 0)
    plsc.subcore_barrier()

    for k in range(ROWS_PER_TILE // SUB):
        row0 = s * ROWS_PER_TILE + k * SUB
        pltpu.sync_copy(acc_sp.at[pl.ds(row0, SUB), :], rb0)
        pltpu.sync_copy(rb0, part_r.at[c, pl.ds(row0, SUB), :])
    if ROWS_PER_TILE % SUB:
        _rem2 = ROWS_PER_TILE % SUB
        row0 = s * ROWS_PER_TILE + ROWS_PER_TILE - _rem2
        pltpu.sync_copy(acc_sp.at[pl.ds(row0, _rem2), :], rb0.at[pl.ds(0, _rem2), :])
        pltpu.sync_copy(rb0.at[pl.ds(0, _rem2), :], part_r.at[c, pl.ds(row0, _rem2), :])


_sc_scatter = pl.kernel(
    _sc_body,
    out_type=jax.ShapeDtypeStruct((NC, OUT_ROWS, D), jnp.float32),
    mesh=plsc.VectorSubcoreMesh(
        core_axis_name="c", subcore_axis_name="s",
        num_cores=NC, num_subcores=NS),
    compiler_params=pltpu.CompilerParams(needs_layout_passes=False),
    scratch_types=[
        pltpu.VMEM_SHARED((CNT_SIZE,), jnp.float32),
        pltpu.VMEM_SHARED((OUT_ROWS, D), jnp.float32),
        pltpu.VMEM((BATCH,), jnp.int32),
        pltpu.VMEM((BATCH,), jnp.int32),
        pltpu.VMEM((BATCH,), jnp.int32),
        pltpu.VMEM((BCH, CHUNK), jnp.int32),
        pltpu.VMEM((NSUB, SUB), jnp.int32),
        pltpu.VMEM((BATCH,), jnp.int32),
        pltpu.VMEM((BATCH,), jnp.float32),
        pltpu.VMEM((BCH, CHUNK), jnp.float32),
        pltpu.VMEM((SUB, D), jnp.float32),
        pltpu.VMEM((SUB, D), jnp.float32),
        pltpu.VMEM((SUB, D), jnp.float32),
        pltpu.VMEM((ZCHUNK,), jnp.float32),
        pltpu.VMEM((CHUNK,), jnp.float32),
        pltpu.SemaphoreType.DMA,
        pltpu.SemaphoreType.DMA,
        pltpu.SemaphoreType.DMA,
    ],
)


def kernel(node_embeds, weight, root, bias, edge_index, edge_type):
    x = node_embeds.astype(jnp.float32)
    src = edge_index[0].astype(jnp.int32)
    dst = edge_index[1].astype(jnp.int32)
    typ = edge_type.astype(jnp.int32)
    pad = E_PAD - E
    src_p = jnp.concatenate([src, jnp.zeros((pad,), jnp.int32)])
    dst_p = jnp.concatenate([dst, jnp.zeros((pad,), jnp.int32)])
    typ_p = jnp.concatenate([typ, jnp.full((pad,), R, jnp.int32)])

    y = _relation_transform(x, weight).reshape(R * N, D)
    partials = _sc_scatter(src_p, dst_p, typ_p, y)
    bias2d = bias.reshape(1, D)
    return _combine(x, root, bias2d, partials)


# speedup vs baseline: 1.0472x; 1.0468x over previous
import functools

import jax
import jax.numpy as jnp
from jax import lax
from jax.experimental import pallas as pl
from jax.experimental.pallas import tpu as pltpu
from jax.experimental.pallas import tpu_sc as plsc

N = 10000
R = 24
D = 128
E = 320000

NC = 2
NS = 16
LANE = 16

EPT = E // (NC * NS)
CHUNK = 128
NCH = (EPT + CHUNK - 1) // CHUNK
SLOTS = NCH * CHUNK
E_PAD = 320256

SEG = R * N
CNT_SIZE = SEG + 128
CNT_PER_TILE = CNT_SIZE // NS

OUT_ROWS = 10112
ROWS_PER_TILE = OUT_ROWS // NS

BLK = 2000
NB = N // BLK


def _ymm_body(x_ref, w_ref, y_ref):
    y_ref[0] = jnp.dot(x_ref[...], w_ref[0], preferred_element_type=jnp.float32)


def _relation_transform(x, w):
    return pl.pallas_call(
        _ymm_body,
        grid=(NB, R),
        in_specs=[
            pl.BlockSpec((BLK, D), lambda b, r: (b, 0)),
            pl.BlockSpec((1, D, D), lambda b, r: (r, 0, 0)),
        ],
        out_specs=pl.BlockSpec((1, BLK, D), lambda b, r: (r, b, 0)),
        out_shape=jax.ShapeDtypeStruct((R, N, D), jnp.float32),
    )(x, w)


def _fin_body(x_ref, root_ref, bias_ref, p0_ref, p1_ref, o_ref):
    x = x_ref[...]
    o_ref[...] = (
        x
        + jnp.dot(x, root_ref[...], preferred_element_type=jnp.float32)
        + bias_ref[...]
        + p0_ref[0]
        + p1_ref[0]
    )


def _combine(x, root, bias2d, partials):
    return pl.pallas_call(
        _fin_body,
        grid=(NB,),
        in_specs=[
            pl.BlockSpec((BLK, D), lambda b: (b, 0)),
            pl.BlockSpec((D, D), lambda b: (0, 0)),
            pl.BlockSpec((1, D), lambda b: (0, 0)),
            pl.BlockSpec((1, BLK, D), lambda b: (0, b, 0)),
            pl.BlockSpec((1, BLK, D), lambda b: (1, b, 0)),
        ],
        out_specs=pl.BlockSpec((BLK, D), lambda b: (b, 0)),
        out_shape=jax.ShapeDtypeStruct((N, D), jnp.float32),
    )(x, root, bias2d, partials, partials)


ZCHUNK = 1072
BATCH = 1024
NBAT = 10
BCH = BATCH // CHUNK
SUB = 64
NSUB = BATCH // SUB


def _sc_body(src_r, dst_r, typ_r, y_r, part_r,
             cnt_sp, acc_sp,
             sstage, dstage, tstage, combA, dstidx, yidxst, normst, cvbuf,
             rb0, rb1, rb2, zlin, onesb, sem, semg, semsc):
    c = lax.axis_index("c")
    s = lax.axis_index("s")
    zero16 = jnp.zeros((LANE,), jnp.float32)
    one16 = jnp.ones((LANE,), jnp.float32)
    lane = lax.broadcasted_iota(jnp.int32, (LANE,), 0)

    def _zrow(i, _):
        for j in range(D // LANE):
            rb0[i, pl.ds(j * LANE, LANE)] = zero16
        return 0
    lax.fori_loop(0, SUB, _zrow, 0)

    def _zlinf(i, _):
        zlin[pl.ds(i * LANE, LANE)] = zero16
        return 0
    lax.fori_loop(0, ZCHUNK // LANE, _zlinf, 0)

    for g in range(CHUNK // LANE):
        onesb[pl.ds(g * LANE, LANE)] = one16

    for k in range(CNT_PER_TILE // ZCHUNK):
        pltpu.sync_copy(
            zlin, cnt_sp.at[pl.ds(s * CNT_PER_TILE + k * ZCHUNK, ZCHUNK)])
    for k in range(ROWS_PER_TILE // SUB):
        pltpu.sync_copy(
            rb0, acc_sp.at[pl.ds(s * ROWS_PER_TILE + k * SUB, SUB), :])
    _rem = ROWS_PER_TILE % SUB
    if _rem:
        pltpu.sync_copy(
            rb0.at[pl.ds(0, _rem), :],
            acc_sp.at[pl.ds(s * ROWS_PER_TILE + ROWS_PER_TILE - _rem, _rem), :])
    plsc.subcore_barrier()

    for p in range(NC):
        base_a = p * (E // NC) + s * EPT

        def _cnt_batch(b, _):
            boff = b * BATCH
            da = pltpu.async_copy(
                typ_r.at[pl.ds(base_a + boff, BATCH)], tstage, sem)
            db = pltpu.async_copy(
                dst_r.at[pl.ds(base_a + boff, BATCH)], dstage, sem)
            da.wait()
            db.wait()

            def _cchunk(ci, _):
                for g in range(CHUNK // LANE):
                    off_l = ci * CHUNK + g * LANE
                    t = tstage[pl.ds(off_l, LANE)]
                    d = dstage[pl.ds(off_l, LANE)]
                    valid = (boff + off_l + lane) < EPT
                    combA[ci, pl.ds(g * LANE, LANE)] = jnp.where(
                        valid, t * N + d, SEG)
                return 0
            lax.fori_loop(0, BCH, _cchunk, 0)
            descs = [
                pltpu.async_copy(
                    onesb, cnt_sp.at[combA.at[ci]], semsc, add=True)
                for ci in range(BCH)
            ]
            for dsc in descs:
                dsc.wait()
            return 0
        with jax.named_scope("phA"):
            lax.fori_loop(0, NBAT, _cnt_batch, 0)
    plsc.subcore_barrier()

    base_c = (s * NC + c) * EPT

    def _edge_batch(b, _):
      boff = b * BATCH
      with jax.named_scope("bidx"):
        d1 = pltpu.async_copy(
            src_r.at[pl.ds(base_c + boff, BATCH)], sstage, sem)
        d2 = pltpu.async_copy(
            dst_r.at[pl.ds(base_c + boff, BATCH)], dstage, sem)
        d3 = pltpu.async_copy(
            typ_r.at[pl.ds(base_c + boff, BATCH)], tstage, sem)
        d1.wait()
        d2.wait()
        d3.wait()

        def _cchunk(ci, _):
            for g in range(CHUNK // LANE):
                off_l = ci * CHUNK + g * LANE
                t = tstage[pl.ds(off_l, LANE)]
                d = dstage[pl.ds(off_l, LANE)]
                valid = (boff + off_l + lane) < EPT
                combA[ci, pl.ds(g * LANE, LANE)] = jnp.where(
                    valid, t * N + d, SEG)
            return 0
        lax.fori_loop(0, BCH, _cchunk, 0)
        gds = [
            pltpu.async_copy(cnt_sp.at[combA.at[ci]], cvbuf.at[ci], sem)
            for ci in range(BCH)
        ]
        for gd in gds:
            gd.wait()

        def _nchunk(ci, _):
            for g in range(CHUNK // LANE):
                off_l = ci * CHUNK + g * LANE
                cv = cvbuf[ci, pl.ds(g * LANE, LANE)]
                t = tstage[pl.ds(off_l, LANE)]
                sv = sstage[pl.ds(off_l, LANE)]
                d = dstage[pl.ds(off_l, LANE)]
                valid = (boff + off_l + lane) < EPT
                normst[pl.ds(off_l, LANE)] = jnp.where(
                    valid, 1.0 / jnp.maximum(cv, 1.0), 0.0)
                yidxst[pl.ds(off_l, LANE)] = jnp.where(valid, t * N + sv, 0)
                k2 = off_l // SUB
                dstidx[k2, pl.ds((off_l % SUB) + 0, LANE)] = jnp.where(
                    valid, d, N)
            return 0
        lax.fori_loop(0, BCH, _nchunk, 0)

      with jax.named_scope("rows"):
        rbs = (rb0, rb1, rb2)
        gd = {}
        sd = {}
        for k in range(min(3, NSUB)):
            gd[k] = pltpu.async_copy(
                y_r.at[yidxst.at[pl.ds(k * SUB, SUB)]], rbs[k % 3], semg)
        for k in range(NSUB):
            cur = rbs[k % 3]
            if k >= 1 and (k + 2) < NSUB:
                sd[k - 1].wait()
                gd[k + 2] = pltpu.async_copy(
                    y_r.at[yidxst.at[pl.ds((k + 2) * SUB, SUB)]],
                    rbs[(k + 2) % 3], semg)
            gd[k].wait()

            def _scale(e, _, _cur=cur, _k=k):
                nb = plsc.load_gather(
                    normst, [jnp.full((LANE,), _k * SUB + e, jnp.int32)])
                for j in range(D // LANE):
                    _cur[e, pl.ds(j * LANE, LANE)] = (
                        _cur[e, pl.ds(j * LANE, LANE)] * nb)
                return 0
            lax.fori_loop(0, SUB, _scale, 0)

            sd[k] = pltpu.async_copy(
                cur, acc_sp.at[dstidx.at[k]], semsc, add=True)
        for k in range(max(0, NSUB - 3), NSUB):
            sd[k].wait()
      return 0
    with jax.named_scope("phBC"):
        lax.fori_loop(0, NBAT, _edge_batch, 0)
    plsc.subcore_barrier()

    for k in range(ROWS_PER_TILE // SUB):
        row0 = s * ROWS_PER_TILE + k * SUB
        pltpu.sync_copy(acc_sp.at[pl.ds(row0, SUB), :], rb0)
        pltpu.sync_copy(rb0, part_r.at[c, pl.ds(row0, SUB), :])
    if ROWS_PER_TILE % SUB:
        _rem2 = ROWS_PER_TILE % SUB
        row0 = s * ROWS_PER_TILE + ROWS_PER_TILE - _rem2
        pltpu.sync_copy(acc_sp.at[pl.ds(row0, _rem2), :], rb0.at[pl.ds(0, _rem2), :])
        pltpu.sync_copy(rb0.at[pl.ds(0, _rem2), :], part_r.at[c, pl.ds(row0, _rem2), :])


_sc_scatter = pl.kernel(
    _sc_body,
    out_type=jax.ShapeDtypeStruct((NC, OUT_ROWS, D), jnp.float32),
    mesh=plsc.VectorSubcoreMesh(
        core_axis_name="c", subcore_axis_name="s",
        num_cores=NC, num_subcores=NS),
    compiler_params=pltpu.CompilerParams(needs_layout_passes=False),
    scratch_types=[
        pltpu.VMEM_SHARED((CNT_SIZE,), jnp.float32),
        pltpu.VMEM_SHARED((OUT_ROWS, D), jnp.float32),
        pltpu.VMEM((BATCH,), jnp.int32),
        pltpu.VMEM((BATCH,), jnp.int32),
        pltpu.VMEM((BATCH,), jnp.int32),
        pltpu.VMEM((BCH, CHUNK), jnp.int32),
        pltpu.VMEM((NSUB, SUB), jnp.int32),
        pltpu.VMEM((BATCH,), jnp.int32),
        pltpu.VMEM((BATCH,), jnp.float32),
        pltpu.VMEM((BCH, CHUNK), jnp.float32),
        pltpu.VMEM((SUB, D), jnp.float32),
        pltpu.VMEM((SUB, D), jnp.float32),
        pltpu.VMEM((SUB, D), jnp.float32),
        pltpu.VMEM((ZCHUNK,), jnp.float32),
        pltpu.VMEM((CHUNK,), jnp.float32),
        pltpu.SemaphoreType.DMA,
        pltpu.SemaphoreType.DMA,
        pltpu.SemaphoreType.DMA,
    ],
)


def kernel(node_embeds, weight, root, bias, edge_index, edge_type):
    x = node_embeds.astype(jnp.float32)
    src = edge_index[0].astype(jnp.int32)
    dst = edge_index[1].astype(jnp.int32)
    typ = edge_type.astype(jnp.int32)
    pad = E_PAD - E
    src_p = jnp.concatenate([src, jnp.zeros((pad,), jnp.int32)])
    dst_p = jnp.concatenate([dst, jnp.zeros((pad,), jnp.int32)])
    typ_p = jnp.concatenate([typ, jnp.full((pad,), R, jnp.int32)])

    y = _relation_transform(x, weight).reshape(R * N, D)
    partials = _sc_scatter(src_p, dst_p, typ_p, y)
    bias2d = bias.reshape(1, D)
    return _combine(x, root, bias2d, partials)
